# baseline (speedup 1.0000x reference)
import numpy as np
import jax
import jax.numpy as jnp
from jax.experimental import pallas as pl

_P, _C, _D, _L, _H, _DH, _FF, _NL = 4, 64, 512, 77, 8, 64, 2048, 2
_NPRE, _NCLS, _NSUF = 4, 2, 3
_EOS = 1 + _NPRE + _NCLS + _NSUF
_CB = 16
_NBLK = _C // _CB
_SO = 6
_SH = 8
_GR = _CB * _SO
_RO = _P * _GR
_RS = _P * _SH
_KC = _SH + _GR
_SCALE = float(1.0 / np.sqrt(_DH))
_F32 = jnp.float32


def _layernorm(h, sc, b):
    m = h.mean(-1, keepdims=True)
    v = ((h - m) ** 2).mean(-1, keepdims=True)
    return (h - m) * jax.lax.rsqrt(v + 1e-5) * sc + b


def _mm(a, b):
    return jax.lax.dot_general(
        a, b, (((a.ndim - 1,), (0,)), ((), ())),
        preferred_element_type=_F32)


def _mmx(a, b):
    return jax.lax.dot_general(
        a, b, (((a.ndim - 1,), (0,)), ((), ())),
        preferred_element_type=_F32,
        precision=jax.lax.Precision.HIGHEST)


def _iota2(shape, dim):
    return jax.lax.broadcasted_iota(jnp.int32, shape, dim)


def _body(cls_ref, stat_ref, shtab_ref,
          ln1s_ref, ln1b_ref, wqkv_ref, bqkv_ref, wo_ref, bo_ref,
          ln2s_ref, ln2b_ref, w1_ref, b1_ref, w2_ref, b2_ref,
          lnfs_ref, lnfb_ref, proj_ref, out_ref):
    r = _iota2((_RS, 32), 0)
    c = _iota2((_RS, 32), 1)
    j, t = r // _SH, r % _SH
    tokrow = jnp.where(t == 0, 0,
                       jnp.where(t <= _NPRE, 1 + j * _NPRE + (t - 1), 17))
    sel_sh = jnp.logical_or(c == tokrow, c == 18 + t).astype(_F32)
    x_sh = _mmx(sel_sh, shtab_ref[...])

    table = jnp.concatenate([cls_ref[...], stat_ref[...]], axis=0)
    r = _iota2((_RO, 48), 0)
    c = _iota2((_RO, 48), 1)
    j = r // _GR
    rem = r % _GR
    cc, t = rem // _SO, rem % _SO
    tokrow = jnp.where(t < _NCLS, cc * _NCLS + t,
                       jnp.where(t < _NCLS + _NSUF, 32 + (t - _NCLS), 35 + j))
    sel = jnp.logical_or(c == tokrow, c == 40 + t).astype(_F32)
    x = _mmx(sel, table)

    r = _iota2((_RS, _RS), 0)
    c = _iota2((_RS, _RS), 1)
    ok = (r // _SH == c // _SH) & (c % _SH <= r % _SH) & (c % _SH <= _NPRE)
    mask_sh = jnp.where(ok, _F32(0.0), _F32(-1e9))

    r = _iota2((_GR, _KC), 0)
    c = _iota2((_GR, _KC), 1)
    co = c - _SH
    ok_own = (c >= _SH) & (co // _SO == r // _SO) & (co % _SO <= r % _SO)
    ok = (c <= _NPRE) | ok_own
    mask = jnp.where(ok, _F32(0.0), _F32(-1e9))[None]

    for l in range(_NL):
        h_sh = _layernorm(x_sh, ln1s_ref[l][None, :], ln1b_ref[l][None, :])
        qkv_sh = _mm(h_sh, wqkv_ref[l]) + bqkv_ref[l][None, :]
        h = _layernorm(x, ln1s_ref[l][None, :], ln1b_ref[l][None, :])
        qkv = _mm(h, wqkv_ref[l]) + bqkv_ref[l][None, :]
        qkv3 = qkv.reshape(_P, _GR, 3 * _D)
        kvcat = jnp.concatenate(
            [qkv_sh.reshape(_P, _SH, 3 * _D), qkv3], axis=1)

        o_cols = []
        for hh in range(_H):
            q = qkv3[:, :, hh * _DH:(hh + 1) * _DH] * _SCALE
            k = kvcat[:, :, _D + hh * _DH:_D + (hh + 1) * _DH]
            v = kvcat[:, :, 2 * _D + hh * _DH:2 * _D + (hh + 1) * _DH]
            s = jax.lax.dot_general(
                q, k, (((2,), (2,)), ((0,), (0,))),
                preferred_element_type=_F32)
            e = jnp.exp(s + mask)
            oh = jax.lax.dot_general(
                e, v, (((2,), (1,)), ((0,), (0,))),
                preferred_element_type=_F32)
            oh = oh / e.sum(-1, keepdims=True)
            o_cols.append(oh.reshape(_RO, _DH))
        o = jnp.concatenate(o_cols, axis=1)
        x = x + _mm(o, wo_ref[l]) + bo_ref[l][None, :]
        h2 = _layernorm(x, ln2s_ref[l][None, :], ln2b_ref[l][None, :])
        g = _mm(h2, w1_ref[l]) + b1_ref[l][None, :]
        g = g * (1.0 / (1.0 + jnp.exp(-1.702 * g)))
        x = x + _mm(g, w2_ref[l]) + b2_ref[l][None, :]

        if l < _NL - 1:
            o_sh_cols = []
            for hh in range(_H):
                q = qkv_sh[:, hh * _DH:(hh + 1) * _DH] * _SCALE
                k = qkv_sh[:, _D + hh * _DH:_D + (hh + 1) * _DH]
                v = qkv_sh[:, 2 * _D + hh * _DH:2 * _D + (hh + 1) * _DH]
                s = jax.lax.dot_general(
                    q, k, (((1,), (1,)), ((), ())),
                    preferred_element_type=_F32)
                e = jnp.exp(s + mask_sh)
                oh = _mm(e, v) / e.sum(-1, keepdims=True)
                o_sh_cols.append(oh)
            o_sh = jnp.concatenate(o_sh_cols, axis=1)
            x_sh = x_sh + _mm(o_sh, wo_ref[l]) + bo_ref[l][None, :]
            h2s = _layernorm(x_sh, ln2s_ref[l][None, :], ln2b_ref[l][None, :])
            gs = _mm(h2s, w1_ref[l]) + b1_ref[l][None, :]
            gs = gs * (1.0 / (1.0 + jnp.exp(-1.702 * gs)))
            x_sh = x_sh + _mm(gs, w2_ref[l]) + b2_ref[l][None, :]

    x = _layernorm(x, lnfs_ref[...], lnfb_ref[...])
    r = _iota2((_CB, _RO), 0)
    c = _iota2((_CB, _RO), 1)
    pick = (c % _SO == _SO - 1) & ((c % _GR) // _SO == r)
    gmat = jnp.where(pick, _F32(1.0 / _P), _F32(0.0))
    fm = _mmx(gmat, x)
    out_ref[...] = _mm(fm, proj_ref[...])[None]


def kernel(sos_token, padding_token, prefix_tokens, class_tokens,
           suffix_tokens, eos_tokens, pos_emb, ln1_scale, ln1_bias, Wqkv,
           bqkv, Wo, bo, ln2_scale, ln2_bias, W1, b1, W2, b2, lnf_scale,
           lnf_bias, text_projection):
    z1 = jnp.zeros((1, _D), _F32)
    stat = jnp.concatenate(
        [suffix_tokens, eos_tokens, z1, pos_emb[5:11],
         jnp.zeros((2, _D), _F32)], axis=0)
    shtab = jnp.concatenate(
        [sos_token[None], prefix_tokens.reshape(_P * _NPRE, _D),
         padding_token[None], pos_emb[:_SH],
         jnp.zeros((32 - 18 - _SH, _D), _F32)], axis=0)
    cls_flat = class_tokens.reshape(_C * _NCLS, _D)
    lnfs2 = lnf_scale[None]
    lnfb2 = lnf_bias[None]

    full = lambda a: pl.BlockSpec(a.shape, lambda i: (0,) * a.ndim)
    operands = (cls_flat, stat, shtab, ln1_scale, ln1_bias, Wqkv, bqkv,
                Wo, bo, ln2_scale, ln2_bias, W1, b1, W2, b2, lnfs2, lnfb2,
                text_projection)
    in_specs = [pl.BlockSpec((_CB * _NCLS, _D), lambda i: (i, 0))]
    in_specs += [full(a) for a in operands[1:]]

    return pl.pallas_call(
        _body,
        grid=(_NBLK,),
        in_specs=in_specs,
        out_specs=pl.BlockSpec((1, _CB, _D), lambda i: (0, i, 0)),
        out_shape=jax.ShapeDtypeStruct((1, _C, _D), _F32),
    )(*operands)


# speedup vs baseline: 1.0204x; 1.0204x over previous
import numpy as np
import jax
import jax.numpy as jnp
from jax.experimental import pallas as pl

_P, _C, _D, _L, _H, _DH, _FF, _NL = 4, 64, 512, 77, 8, 64, 2048, 2
_NPRE, _NCLS, _NSUF = 4, 2, 3
_EOS = 1 + _NPRE + _NCLS + _NSUF
_CB = 16
_NBLK = _C // _CB
_SO = 6
_SH = 8
_GR = _CB * _SO
_RO = _P * _GR
_RS = _P * _SH
_KC = _SH + _GR
_SCALE = float(1.0 / np.sqrt(_DH))
_F32 = jnp.float32


def _layernorm(h, sc, b):
    m = h.mean(-1, keepdims=True)
    v = ((h - m) ** 2).mean(-1, keepdims=True)
    return (h - m) * jax.lax.rsqrt(v + 1e-5) * sc + b


def _mm(a, b):
    return jax.lax.dot_general(
        a, b, (((a.ndim - 1,), (0,)), ((), ())),
        preferred_element_type=_F32)


def _mmx(a, b):
    return jax.lax.dot_general(
        a, b, (((a.ndim - 1,), (0,)), ((), ())),
        preferred_element_type=_F32,
        precision=jax.lax.Precision.HIGHEST)


def _iota2(shape, dim):
    return jax.lax.broadcasted_iota(jnp.int32, shape, dim)


def _body(cls_ref, stat_ref, shtab_ref,
          ln1s_ref, ln1b_ref, wqkv_ref, bqkv_ref, wo_ref, bo_ref,
          ln2s_ref, ln2b_ref, w1_ref, b1_ref, w2_ref, b2_ref,
          lnfs_ref, lnfb_ref, proj_ref, out_ref):
    r = _iota2((_RS, 32), 0)
    c = _iota2((_RS, 32), 1)
    j, t = r // _SH, r % _SH
    tokrow = jnp.where(t == 0, 0,
                       jnp.where(t <= _NPRE, 1 + j * _NPRE + (t - 1), 17))
    sel_sh = jnp.logical_or(c == tokrow, c == 18 + t).astype(_F32)
    x_sh = _mmx(sel_sh, shtab_ref[...])

    table = jnp.concatenate([cls_ref[...], stat_ref[...]], axis=0)
    r = _iota2((_RO, 48), 0)
    c = _iota2((_RO, 48), 1)
    j = r // _GR
    rem = r % _GR
    cc, t = rem // _SO, rem % _SO
    tokrow = jnp.where(t < _NCLS, cc * _NCLS + t,
                       jnp.where(t < _NCLS + _NSUF, 32 + (t - _NCLS), 35 + j))
    sel = jnp.logical_or(c == tokrow, c == 40 + t).astype(_F32)
    x = _mmx(sel, table)

    r = _iota2((_RS, _RS), 0)
    c = _iota2((_RS, _RS), 1)
    ok = (r // _SH == c // _SH) & (c % _SH <= r % _SH) & (c % _SH <= _NPRE)
    mask_sh = jnp.where(ok, _F32(0.0), _F32(-1e9))

    r = _iota2((_GR, _KC), 0)
    c = _iota2((_GR, _KC), 1)
    co = c - _SH
    ok_own = (c >= _SH) & (co // _SO == r // _SO) & (co % _SO <= r % _SO)
    ok = (c <= _NPRE) | ok_own
    mask = jnp.where(ok, _F32(0.0), _F32(-1e9))[None]

    for l in range(_NL):
        h_sh = _layernorm(x_sh, ln1s_ref[l][None, :], ln1b_ref[l][None, :])
        qkv_sh = _mm(h_sh, wqkv_ref[l]) + bqkv_ref[l][None, :]
        h = _layernorm(x, ln1s_ref[l][None, :], ln1b_ref[l][None, :])
        qkv = _mm(h, wqkv_ref[l]) + bqkv_ref[l][None, :]
        qkv3 = qkv.reshape(_P, _GR, 3 * _D)
        kvcat = jnp.concatenate(
            [qkv_sh.reshape(_P, _SH, 3 * _D), qkv3], axis=1)

        o_cols = []
        for hh in range(_H):
            q = qkv3[:, :, hh * _DH:(hh + 1) * _DH] * _SCALE
            k = kvcat[:, :, _D + hh * _DH:_D + (hh + 1) * _DH]
            v = kvcat[:, :, 2 * _D + hh * _DH:2 * _D + (hh + 1) * _DH]
            s = jax.lax.dot_general(
                q, k, (((2,), (2,)), ((0,), (0,))),
                preferred_element_type=_F32)
            e = jnp.exp(s + mask)
            oh = jax.lax.dot_general(
                e, v, (((2,), (1,)), ((0,), (0,))),
                preferred_element_type=_F32)
            oh = oh / e.sum(-1, keepdims=True)
            o_cols.append(oh.reshape(_RO, _DH))
        o = jnp.concatenate(o_cols, axis=1)
        x = x + _mm(o, wo_ref[l]) + bo_ref[l][None, :]
        h2 = _layernorm(x, ln2s_ref[l][None, :], ln2b_ref[l][None, :])
        g = _mm(h2, w1_ref[l]) + b1_ref[l][None, :]
        g = g * (1.0 / (1.0 + jnp.exp(-1.702 * g)))
        x = x + _mm(g, w2_ref[l]) + b2_ref[l][None, :]

        if l < _NL - 1:
            o_sh_cols = []
            for hh in range(_H):
                q = qkv_sh[:, hh * _DH:(hh + 1) * _DH] * _SCALE
                k = qkv_sh[:, _D + hh * _DH:_D + (hh + 1) * _DH]
                v = qkv_sh[:, 2 * _D + hh * _DH:2 * _D + (hh + 1) * _DH]
                s = jax.lax.dot_general(
                    q, k, (((1,), (1,)), ((), ())),
                    preferred_element_type=_F32)
                e = jnp.exp(s + mask_sh)
                oh = _mm(e, v) / e.sum(-1, keepdims=True)
                o_sh_cols.append(oh)
            o_sh = jnp.concatenate(o_sh_cols, axis=1)
            x_sh = x_sh + _mm(o_sh, wo_ref[l]) + bo_ref[l][None, :]
            h2s = _layernorm(x_sh, ln2s_ref[l][None, :], ln2b_ref[l][None, :])
            gs = _mm(h2s, w1_ref[l]) + b1_ref[l][None, :]
            gs = gs * (1.0 / (1.0 + jnp.exp(-1.702 * gs)))
            x_sh = x_sh + _mm(gs, w2_ref[l]) + b2_ref[l][None, :]

    x = _layernorm(x, lnfs_ref[...], lnfb_ref[...])
    r = _iota2((_CB, _RO), 0)
    c = _iota2((_CB, _RO), 1)
    pick = (c % _SO == _SO - 1) & ((c % _GR) // _SO == r)
    gmat = jnp.where(pick, _F32(1.0 / _P), _F32(0.0))
    fm = _mm(gmat, x)
    out_ref[...] = _mm(fm, proj_ref[...])[None]


def kernel(sos_token, padding_token, prefix_tokens, class_tokens,
           suffix_tokens, eos_tokens, pos_emb, ln1_scale, ln1_bias, Wqkv,
           bqkv, Wo, bo, ln2_scale, ln2_bias, W1, b1, W2, b2, lnf_scale,
           lnf_bias, text_projection):
    z1 = jnp.zeros((1, _D), _F32)
    stat = jnp.concatenate(
        [suffix_tokens, eos_tokens, z1, pos_emb[5:11],
         jnp.zeros((2, _D), _F32)], axis=0)
    shtab = jnp.concatenate(
        [sos_token[None], prefix_tokens.reshape(_P * _NPRE, _D),
         padding_token[None], pos_emb[:_SH],
         jnp.zeros((32 - 18 - _SH, _D), _F32)], axis=0)
    cls_flat = class_tokens.reshape(_C * _NCLS, _D)
    lnfs2 = lnf_scale[None]
    lnfb2 = lnf_bias[None]

    full = lambda a: pl.BlockSpec(a.shape, lambda i: (0,) * a.ndim)
    operands = (cls_flat, stat, shtab, ln1_scale, ln1_bias, Wqkv, bqkv,
                Wo, bo, ln2_scale, ln2_bias, W1, b1, W2, b2, lnfs2, lnfb2,
                text_projection)
    in_specs = [pl.BlockSpec((_CB * _NCLS, _D), lambda i: (i, 0))]
    in_specs += [full(a) for a in operands[1:]]

    return pl.pallas_call(
        _body,
        grid=(_NBLK,),
        in_specs=in_specs,
        out_specs=pl.BlockSpec((1, _CB, _D), lambda i: (0, i, 0)),
        out_shape=jax.ShapeDtypeStruct((1, _C, _D), _F32),
    )(*operands)


# speedup vs baseline: 1.2107x; 1.1865x over previous
import numpy as np
import jax
import jax.numpy as jnp
from jax.experimental import pallas as pl

_P, _C, _D, _L, _H, _DH, _FF, _NL = 4, 64, 512, 77, 8, 64, 2048, 2
_NPRE, _NCLS, _NSUF = 4, 2, 3
_EOS = 1 + _NPRE + _NCLS + _NSUF
_CB = 32
_NBLK = _C // _CB
_SO = 6
_SH = 8
_GR = 16 * _SO
_GN = _P * (_CB // 16)
_CHUNK = _CB * _SO
_CBT = _CB * _NCLS
_RO = _P * _CHUNK
_RS = _P * _SH
_KC = _SH + _GR
_SCALE = float(1.0 / np.sqrt(_DH))
_F32 = jnp.float32


def _layernorm(h, sc, b):
    m = h.mean(-1, keepdims=True)
    v = ((h - m) ** 2).mean(-1, keepdims=True)
    return (h - m) * jax.lax.rsqrt(v + 1e-5) * sc + b


def _mm(a, b):
    return jax.lax.dot_general(
        a, b, (((a.ndim - 1,), (0,)), ((), ())),
        preferred_element_type=_F32)


def _mmx(a, b):
    return jax.lax.dot_general(
        a, b, (((a.ndim - 1,), (0,)), ((), ())),
        preferred_element_type=_F32,
        precision=jax.lax.Precision.HIGHEST)


def _iota2(shape, dim):
    return jax.lax.broadcasted_iota(jnp.int32, shape, dim)


def _body(cls_ref, stat_ref, shtab_ref,
          ln1s_ref, ln1b_ref, wqkv_ref, bqkv_ref, wo_ref, bo_ref,
          ln2s_ref, ln2b_ref, w1_ref, b1_ref, w2_ref, b2_ref,
          lnfs_ref, lnfb_ref, proj_ref, out_ref):
    r = _iota2((_RS, 32), 0)
    c = _iota2((_RS, 32), 1)
    j, t = r // _SH, r % _SH
    tokrow = jnp.where(t == 0, 0,
                       jnp.where(t <= _NPRE, 1 + j * _NPRE + (t - 1), 17))
    sel_sh = jnp.logical_or(c == tokrow, c == 18 + t).astype(_F32)
    x_sh = _mmx(sel_sh, shtab_ref[...])

    table = jnp.concatenate([cls_ref[...], stat_ref[...]], axis=0)
    r = _iota2((_RO, _CBT + 16), 0)
    c = _iota2((_RO, _CBT + 16), 1)
    j = r // _CHUNK
    rem = r % _CHUNK
    cc, t = rem // _SO, rem % _SO
    tokrow = jnp.where(t < _NCLS, cc * _NCLS + t,
                       jnp.where(t < _NCLS + _NSUF, _CBT + (t - _NCLS),
                                 _CBT + 3 + j))
    sel = jnp.logical_or(c == tokrow, c == _CBT + 8 + t).astype(_F32)
    x = _mmx(sel, table)

    r = _iota2((_RS, _RS), 0)
    c = _iota2((_RS, _RS), 1)
    ok = (r // _SH == c // _SH) & (c % _SH <= r % _SH) & (c % _SH <= _NPRE)
    mask_sh = jnp.where(ok, _F32(0.0), _F32(-1e9))

    r = _iota2((_GR, _KC), 0)
    c = _iota2((_GR, _KC), 1)
    co = c - _SH
    ok_own = (c >= _SH) & (co // _SO == r // _SO) & (co % _SO <= r % _SO)
    ok = (c <= _NPRE) | ok_own
    mask = jnp.where(ok, _F32(0.0), _F32(-1e9))[None]

    for l in range(_NL):
        h_sh = _layernorm(x_sh, ln1s_ref[l][None, :], ln1b_ref[l][None, :])
        qkv_sh = _mm(h_sh, wqkv_ref[l]) + bqkv_ref[l][None, :]
        h = _layernorm(x, ln1s_ref[l][None, :], ln1b_ref[l][None, :])
        qkv = _mm(h, wqkv_ref[l]) + bqkv_ref[l][None, :]
        qkv3 = qkv.reshape(_GN, _GR, 3 * _D)
        shkv = jnp.broadcast_to(
            qkv_sh.reshape(_P, 1, _SH, 3 * _D),
            (_P, _GN // _P, _SH, 3 * _D)).reshape(_GN, _SH, 3 * _D)
        kvcat = jnp.concatenate([shkv, qkv3], axis=1)

        o_cols = []
        for hh in range(_H):
            q = qkv3[:, :, hh * _DH:(hh + 1) * _DH] * _SCALE
            k = kvcat[:, :, _D + hh * _DH:_D + (hh + 1) * _DH]
            v = kvcat[:, :, 2 * _D + hh * _DH:2 * _D + (hh + 1) * _DH]
            s = jax.lax.dot_general(
                q, k, (((2,), (2,)), ((0,), (0,))),
                preferred_element_type=_F32)
            e = jnp.exp(s + mask)
            oh = jax.lax.dot_general(
                e, v, (((2,), (1,)), ((0,), (0,))),
                preferred_element_type=_F32)
            oh = oh / e.sum(-1, keepdims=True)
            o_cols.append(oh.reshape(_RO, _DH))
        o = jnp.concatenate(o_cols, axis=1)
        x = x + _mm(o, wo_ref[l]) + bo_ref[l][None, :]
        h2 = _layernorm(x, ln2s_ref[l][None, :], ln2b_ref[l][None, :])
        g = _mm(h2, w1_ref[l]) + b1_ref[l][None, :]
        g = g * (1.0 / (1.0 + jnp.exp(-1.702 * g)))
        x = x + _mm(g, w2_ref[l]) + b2_ref[l][None, :]

        if l < _NL - 1:
            o_sh_cols = []
            for hh in range(_H):
                q = qkv_sh[:, hh * _DH:(hh + 1) * _DH] * _SCALE
                k = qkv_sh[:, _D + hh * _DH:_D + (hh + 1) * _DH]
                v = qkv_sh[:, 2 * _D + hh * _DH:2 * _D + (hh + 1) * _DH]
                s = jax.lax.dot_general(
                    q, k, (((1,), (1,)), ((), ())),
                    preferred_element_type=_F32)
                e = jnp.exp(s + mask_sh)
                oh = _mm(e, v) / e.sum(-1, keepdims=True)
                o_sh_cols.append(oh)
            o_sh = jnp.concatenate(o_sh_cols, axis=1)
            x_sh = x_sh + _mm(o_sh, wo_ref[l]) + bo_ref[l][None, :]
            h2s = _layernorm(x_sh, ln2s_ref[l][None, :], ln2b_ref[l][None, :])
            gs = _mm(h2s, w1_ref[l]) + b1_ref[l][None, :]
            gs = gs * (1.0 / (1.0 + jnp.exp(-1.702 * gs)))
            x_sh = x_sh + _mm(gs, w2_ref[l]) + b2_ref[l][None, :]

    x = _layernorm(x, lnfs_ref[...], lnfb_ref[...])
    r = _iota2((_CB, _RO), 0)
    c = _iota2((_CB, _RO), 1)
    pick = (c % _SO == _SO - 1) & ((c % _CHUNK) // _SO == r)
    gmat = jnp.where(pick, _F32(1.0 / _P), _F32(0.0))
    fm = _mm(gmat, x)
    out_ref[...] = _mm(fm, proj_ref[...])[None]


def kernel(sos_token, padding_token, prefix_tokens, class_tokens,
           suffix_tokens, eos_tokens, pos_emb, ln1_scale, ln1_bias, Wqkv,
           bqkv, Wo, bo, ln2_scale, ln2_bias, W1, b1, W2, b2, lnf_scale,
           lnf_bias, text_projection):
    z1 = jnp.zeros((1, _D), _F32)
    stat = jnp.concatenate(
        [suffix_tokens, eos_tokens, z1, pos_emb[5:11],
         jnp.zeros((2, _D), _F32)], axis=0)
    shtab = jnp.concatenate(
        [sos_token[None], prefix_tokens.reshape(_P * _NPRE, _D),
         padding_token[None], pos_emb[:_SH],
         jnp.zeros((32 - 18 - _SH, _D), _F32)], axis=0)
    cls_flat = class_tokens.reshape(_C * _NCLS, _D)
    lnfs2 = lnf_scale[None]
    lnfb2 = lnf_bias[None]

    full = lambda a: pl.BlockSpec(a.shape, lambda i: (0,) * a.ndim)
    operands = (cls_flat, stat, shtab, ln1_scale, ln1_bias, Wqkv, bqkv,
                Wo, bo, ln2_scale, ln2_bias, W1, b1, W2, b2, lnfs2, lnfb2,
                text_projection)
    in_specs = [pl.BlockSpec((_CB * _NCLS, _D), lambda i: (i, 0))]
    in_specs += [full(a) for a in operands[1:]]

    return pl.pallas_call(
        _body,
        grid=(_NBLK,),
        in_specs=in_specs,
        out_specs=pl.BlockSpec((1, _CB, _D), lambda i: (0, i, 0)),
        out_shape=jax.ShapeDtypeStruct((1, _C, _D), _F32),
    )(*operands)
